# Initial kernel scaffold; baseline (speedup 1.0000x reference)
import jax
import jax.numpy as jnp
from jax.experimental import pallas as pl


def kernel(x, parents_mask, available_actions, father_action_weights, W, b, deterministic):
    raise NotImplementedError("write your pallas kernel here")


# speedup vs baseline: 2.6144x; 2.6144x over previous
import jax
import jax.numpy as jnp
from jax.experimental import pallas as pl
from jax.experimental.pallas import tpu as pltpu

_N = 32
_A = 16
_XD = 512
_ROW_BLK = 256


def _head_body(x_ref, pmf_ref, wrow_ref, w1_ref, w2_ref, b_ref, avail_ref,
               act_ref, alp_ref):
    x = x_ref[...]
    pmfs = pmf_ref[...] * wrow_ref[...]
    logits = jnp.dot(x, w1_ref[...], preferred_element_type=jnp.float32)
    logits = logits + jnp.dot(pmfs, w2_ref[...], preferred_element_type=jnp.float32)
    logits = logits + b_ref[...]
    masked = jnp.where(avail_ref[...] > 0, logits, -1e10)
    m = jnp.max(masked, axis=-1, keepdims=True)
    lse = jnp.log(jnp.sum(jnp.exp(masked - m), axis=-1, keepdims=True))
    act_ref[...] = jnp.argmax(masked, axis=-1, keepdims=True).astype(jnp.int32)
    alp_ref[...] = -lse


def _father_body(pmf_ref, wv_ref, e_ref, out_ref):
    scaled = pmf_ref[...] * wv_ref[...]
    out_ref[...] = jnp.dot(scaled, e_ref[...], preferred_element_type=jnp.float32)


def kernel(x, parents_mask, available_actions, father_action_weights, W, b,
           deterministic=True):
    T = x.shape[0]
    n = _N
    A = _A
    nn = n * n
    pmf2d = parents_mask.reshape(T, nn).astype(jnp.float32)
    wvec = jnp.tile(father_action_weights, n)
    W1 = W[:_XD]
    W2 = W[_XD::A]

    q = jnp.arange(2 * nn)
    s = jnp.arange(128)
    E = ((q[None, :] % A == 0) & (q[None, :] // A == s[:, None])).astype(jnp.float32)

    father = pl.pallas_call(
        _father_body,
        grid=(T // _ROW_BLK, nn // 128),
        in_specs=[
            pl.BlockSpec((_ROW_BLK, 128), lambda r, c: (r, c)),
            pl.BlockSpec((1, 128), lambda r, c: (0, c)),
            pl.BlockSpec((128, 2 * nn), lambda r, c: (0, 0)),
        ],
        out_specs=pl.BlockSpec((_ROW_BLK, 2 * nn), lambda r, c: (r, c)),
        out_shape=jax.ShapeDtypeStruct((T, nn * A), jnp.float32),
    )(pmf2d, wvec.reshape(1, nn), E)

    actions, alp = pl.pallas_call(
        _head_body,
        in_specs=[pl.BlockSpec(memory_space=pltpu.VMEM)] * 7,
        out_specs=[pl.BlockSpec(memory_space=pltpu.VMEM)] * 2,
        out_shape=[
            jax.ShapeDtypeStruct((T, 1), jnp.int32),
            jax.ShapeDtypeStruct((T, 1), jnp.float32),
        ],
    )(x, pmf2d, wvec.reshape(1, nn), W1, W2, b.reshape(1, A),
      available_actions)

    return (actions, alp, father)
